# Initial kernel scaffold; baseline (speedup 1.0000x reference)
import functools

import jax
import jax.numpy as jnp
from jax import lax
from jax.experimental import pallas as pl
from jax.experimental.pallas import tpu as pltpu
from jax.experimental.pallas import tpu_sc as plsc

_NC = 2
_NS = 16
_NW = _NC * _NS


@functools.partial(jax.jit, static_argnames=("chunk",))
def _gather_sc(flat_idx, table, chunk=512):
    n = flat_idx.shape[0]
    d = table.shape[1]
    n_per_w = n // _NW
    n_chunks = n_per_w // chunk
    assert n_per_w % chunk == 0

    mesh = plsc.VectorSubcoreMesh(core_axis_name="c", subcore_axis_name="s")

    @functools.partial(
        pl.kernel,
        out_type=jax.ShapeDtypeStruct((n, d), jnp.float32),
        mesh=mesh,
        scratch_types=[
            pltpu.VMEM((n_per_w,), jnp.int32),
            pltpu.VMEM((2, chunk, d), jnp.float32),
            pltpu.SemaphoreType.DMA,
            pltpu.SemaphoreType.DMA,
        ],
    )
    def k(idx_hbm, table_hbm, out_hbm, idx_v, rows_v, gsem, osem):
        wid = lax.axis_index("s") * _NC + lax.axis_index("c")
        base = wid * n_per_w
        pltpu.sync_copy(idx_hbm.at[pl.ds(base, n_per_w)], idx_v)

        pltpu.async_copy(
            table_hbm.at[idx_v.at[pl.ds(0, chunk)]], rows_v.at[0], gsem
        )

        def body(g, carry):
            buf = lax.rem(g, 2)
            nxt = lax.rem(g + 1, 2)

            pltpu.make_async_copy(
                table_hbm.at[idx_v.at[pl.ds(g * chunk, chunk)]],
                rows_v.at[buf],
                gsem,
            ).wait()

            @pl.when(g >= 2)
            def _():
                pltpu.make_async_copy(
                    rows_v.at[nxt],
                    out_hbm.at[pl.ds(base + (g - 1) * chunk, chunk)],
                    osem,
                ).wait()

            @pl.when(g + 1 < n_chunks)
            def _():
                pltpu.async_copy(
                    table_hbm.at[idx_v.at[pl.ds((g + 1) * chunk, chunk)]],
                    rows_v.at[nxt],
                    gsem,
                )

            pltpu.async_copy(
                rows_v.at[buf],
                out_hbm.at[pl.ds(base + g * chunk, chunk)],
                osem,
            )
            return carry

        lax.fori_loop(0, n_chunks, body, 0)

        pltpu.make_async_copy(
            rows_v.at[0],
            out_hbm.at[pl.ds(base + (n_chunks - 2) * chunk, chunk)],
            osem,
        ).wait()
        pltpu.make_async_copy(
            rows_v.at[0],
            out_hbm.at[pl.ds(base + (n_chunks - 1) * chunk, chunk)],
            osem,
        ).wait()

    return k(flat_idx, table)


def kernel(entity_indices, table):
    b, h = entity_indices.shape
    flat_idx = entity_indices.reshape(b * h).astype(jnp.int32)
    out = _gather_sc(flat_idx, table)
    return out.reshape(b, h, table.shape[1])


# speedup vs baseline: 1.8722x; 1.8722x over previous
import functools

import jax
import jax.numpy as jnp
from jax import lax
from jax.experimental import pallas as pl
from jax.experimental.pallas import tpu as pltpu
from jax.experimental.pallas import tpu_sc as plsc

_NC = 2
_NS = 16
_NW = _NC * _NS


@functools.partial(jax.jit, static_argnames=("chunk",))
def _gather_sc(flat_idx, table, chunk=512):
    n = flat_idx.shape[0]
    d = table.shape[1]
    n_per_w = n // _NW
    n_chunks = n_per_w // chunk
    assert n_per_w % chunk == 0

    mesh = plsc.VectorSubcoreMesh(core_axis_name="c", subcore_axis_name="s")

    @functools.partial(
        pl.kernel,
        out_type=jax.ShapeDtypeStruct((n, d), jnp.float32),
        mesh=mesh,
        scratch_types=[
            pltpu.VMEM((n_per_w,), jnp.int32),
            pltpu.VMEM((2, chunk, d), jnp.float32),
            pltpu.SemaphoreType.DMA,
            pltpu.SemaphoreType.DMA,
        ],
        compiler_params=pltpu.CompilerParams(use_tc_tiling_on_sc=False),
    )
    def k(idx_hbm, table_hbm, out_hbm, idx_v, rows_v, gsem, osem):
        wid = lax.axis_index("s") * _NC + lax.axis_index("c")
        base = wid * n_per_w
        pltpu.sync_copy(idx_hbm.at[pl.ds(base, n_per_w)], idx_v)

        pltpu.async_copy(
            table_hbm.at[idx_v.at[pl.ds(0, chunk)]], rows_v.at[0], gsem
        )

        def body(g, carry):
            buf = lax.rem(g, 2)
            nxt = lax.rem(g + 1, 2)

            pltpu.make_async_copy(
                table_hbm.at[idx_v.at[pl.ds(g * chunk, chunk)]],
                rows_v.at[buf],
                gsem,
            ).wait()

            @pl.when(g >= 2)
            def _():
                pltpu.make_async_copy(
                    rows_v.at[nxt],
                    out_hbm.at[pl.ds(base + (g - 1) * chunk, chunk)],
                    osem,
                ).wait()

            @pl.when(g + 1 < n_chunks)
            def _():
                pltpu.async_copy(
                    table_hbm.at[idx_v.at[pl.ds((g + 1) * chunk, chunk)]],
                    rows_v.at[nxt],
                    gsem,
                )

            pltpu.async_copy(
                rows_v.at[buf],
                out_hbm.at[pl.ds(base + g * chunk, chunk)],
                osem,
            )
            return carry

        lax.fori_loop(0, n_chunks, body, 0)

        pltpu.make_async_copy(
            rows_v.at[0],
            out_hbm.at[pl.ds(base + (n_chunks - 2) * chunk, chunk)],
            osem,
        ).wait()
        pltpu.make_async_copy(
            rows_v.at[0],
            out_hbm.at[pl.ds(base + (n_chunks - 1) * chunk, chunk)],
            osem,
        ).wait()

    return k(flat_idx, table)


def kernel(entity_indices, table):
    b, h = entity_indices.shape
    flat_idx = entity_indices.reshape(b * h).astype(jnp.int32)
    out = _gather_sc(flat_idx, table)
    return out.reshape(b, h, table.shape[1])


# speedup vs baseline: 1.8778x; 1.0030x over previous
import functools

import jax
import jax.numpy as jnp
from jax import lax
from jax.experimental import pallas as pl
from jax.experimental.pallas import tpu as pltpu
from jax.experimental.pallas import tpu_sc as plsc

_NC = 2
_NS = 16
_NW = _NC * _NS


@functools.partial(jax.jit, static_argnames=("chunk", "nbuf", "depth"))
def _gather_sc(flat_idx, table, chunk=320, nbuf=4, depth=2):
    n = flat_idx.shape[0]
    d = table.shape[1]
    n_per_w = n // _NW
    n_chunks = n_per_w // chunk
    assert n_per_w % chunk == 0 and n_chunks > nbuf

    mesh = plsc.VectorSubcoreMesh(core_axis_name="c", subcore_axis_name="s")

    @functools.partial(
        pl.kernel,
        out_type=jax.ShapeDtypeStruct((n, d), jnp.float32),
        mesh=mesh,
        scratch_types=[
            pltpu.VMEM((n_per_w,), jnp.int32),
            pltpu.VMEM((nbuf, chunk, d), jnp.float32),
            pltpu.SemaphoreType.DMA,
            pltpu.SemaphoreType.DMA,
        ],
        compiler_params=pltpu.CompilerParams(use_tc_tiling_on_sc=False),
    )
    def k(idx_hbm, table_hbm, out_hbm, idx_v, rows_v, gsem, osem):
        wid = lax.axis_index("s") * _NC + lax.axis_index("c")
        base = wid * n_per_w
        pltpu.sync_copy(idx_hbm.at[pl.ds(base, n_per_w)], idx_v)

        def gather(g, buf):
            pltpu.async_copy(
                table_hbm.at[idx_v.at[pl.ds(g * chunk, chunk)]],
                rows_v.at[buf],
                gsem,
            )

        def out_copy(g, buf):
            return pltpu.make_async_copy(
                rows_v.at[buf],
                out_hbm.at[pl.ds(base + g * chunk, chunk)],
                osem,
            )

        for g in range(depth):
            gather(g, g % nbuf)

        def body(g, carry):
            buf = lax.rem(g, nbuf)
            pltpu.make_async_copy(
                table_hbm.at[idx_v.at[pl.ds(g * chunk, chunk)]],
                rows_v.at[buf],
                gsem,
            ).wait()

            @pl.when(g >= nbuf - depth)
            def _():
                out_copy(g + depth - nbuf, lax.rem(g + depth, nbuf)).wait()

            @pl.when(g + depth < n_chunks)
            def _():
                gather(g + depth, lax.rem(g + depth, nbuf))

            out_copy(g, buf).start()
            return carry

        lax.fori_loop(0, n_chunks, body, 0)

        for t in range(nbuf - depth):
            out_copy(n_chunks - (nbuf - depth) + t, 0).wait()

    return k(flat_idx, table)


def kernel(entity_indices, table):
    b, h = entity_indices.shape
    flat_idx = entity_indices.reshape(b * h).astype(jnp.int32)
    out = _gather_sc(flat_idx, table)
    return out.reshape(b, h, table.shape[1])


# speedup vs baseline: 1.8853x; 1.0040x over previous
import functools

import jax
import jax.numpy as jnp
from jax import lax
from jax.experimental import pallas as pl
from jax.experimental.pallas import tpu as pltpu
from jax.experimental.pallas import tpu_sc as plsc

_NC = 2
_NS = 16
_NW = _NC * _NS


@functools.partial(jax.jit, static_argnames=("b_per_chunk", "nbuf", "depth"))
def _gather_sc(idx, table, b_per_chunk=8, nbuf=4, depth=2):
    batch, hist = idx.shape
    d = table.shape[1]
    b_per_w = batch // _NW
    n_chunks = b_per_w // b_per_chunk
    assert b_per_w % b_per_chunk == 0 and n_chunks > nbuf

    mesh = plsc.VectorSubcoreMesh(core_axis_name="c", subcore_axis_name="s")

    @functools.partial(
        pl.kernel,
        out_type=jax.ShapeDtypeStruct((batch, hist, d), jnp.float32),
        mesh=mesh,
        scratch_types=[
            pltpu.VMEM((b_per_w, hist), jnp.int32),
            pltpu.VMEM((nbuf, b_per_chunk, hist, d), jnp.float32),
            pltpu.SemaphoreType.DMA,
            pltpu.SemaphoreType.DMA,
        ],
        compiler_params=pltpu.CompilerParams(use_tc_tiling_on_sc=False),
    )
    def k(idx_hbm, table_hbm, out_hbm, idx_v, rows_v, gsem, osem):
        wid = lax.axis_index("s") * _NC + lax.axis_index("c")
        base_b = wid * b_per_w
        pltpu.sync_copy(idx_hbm.at[pl.ds(base_b, b_per_w)], idx_v)

        def gather(g, buf):
            for j in range(b_per_chunk):
                pltpu.async_copy(
                    table_hbm.at[idx_v.at[g * b_per_chunk + j]],
                    rows_v.at[buf, j],
                    gsem,
                )

        def gather_wait(g, buf):
            for j in range(b_per_chunk):
                pltpu.make_async_copy(
                    table_hbm.at[idx_v.at[g * b_per_chunk + j]],
                    rows_v.at[buf, j],
                    gsem,
                ).wait()

        def out_copy(g, buf):
            return pltpu.make_async_copy(
                rows_v.at[buf],
                out_hbm.at[pl.ds(base_b + g * b_per_chunk, b_per_chunk)],
                osem,
            )

        for g in range(depth):
            gather(g, g % nbuf)

        def body(g, carry):
            buf = lax.rem(g, nbuf)
            gather_wait(g, buf)

            @pl.when(g >= nbuf - depth)
            def _():
                out_copy(g + depth - nbuf, lax.rem(g + depth, nbuf)).wait()

            @pl.when(g + depth < n_chunks)
            def _():
                gather(g + depth, lax.rem(g + depth, nbuf))

            out_copy(g, buf).start()
            return carry

        lax.fori_loop(0, n_chunks, body, 0)

        for t in range(nbuf - depth):
            out_copy(n_chunks - (nbuf - depth) + t, 0).wait()

    return k(idx, table)


def kernel(entity_indices, table):
    return _gather_sc(entity_indices.astype(jnp.int32), table)
